# baseline (speedup 1.0000x reference)
import functools

import jax
import jax.numpy as jnp
from jax import lax
from jax.experimental import pallas as pl
from jax.experimental.pallas import tpu as pltpu
from jax.experimental.pallas import tpu_sc as plsc

DN = ('NCHW', 'OIHW', 'NCHW')

N_TOK = 4096
D_EMB = 64
K_CB = 8192
TN = 1024
TK = 8192
NI = N_TOK // TN
NJ = K_CB // TK


def _conv(x, w, b, stride, pad):
    y = lax.conv_general_dilated(x, w, (stride, stride),
                                 ((pad, pad), (pad, pad)),
                                 dimension_numbers=DN)
    return y + b[None, :, None, None]


def _convT(x, w, b):
    y = lax.conv_transpose(x, w, (2, 2), 'SAME', dimension_numbers=DN)
    return y + b[None, :, None, None]


def _vq_body(zf_ref, cbt_ref, idx_ref, loss_ref, best_ref, besti_ref):
    j = pl.program_id(1)

    @pl.when(j == 0)
    def _():
        best_ref[...] = jnp.full((TN, 1), jnp.inf, jnp.float32)
        besti_ref[...] = jnp.zeros((TN, 1), jnp.int32)

    zf = zf_ref[...]
    cbt = cbt_ref[...]
    zn = jnp.sum(zf * zf, axis=1, keepdims=True)
    cn = jnp.sum(cbt * cbt, axis=0, keepdims=True)
    mm2 = lax.dot_general(zf * (-2.0), cbt, (((1,), (0,)), ((), ())),
                          preferred_element_type=jnp.float32)
    d = (zn + cn) + mm2
    lmin = jnp.min(d, axis=1, keepdims=True)
    ids = lax.broadcasted_iota(jnp.int32, (TN, TK), 1).astype(jnp.float32)
    larg_f = jnp.min(jnp.where(d == lmin, ids, float(K_CB)), axis=1,
                     keepdims=True)
    larg = larg_f.astype(jnp.int32) + j * TK
    take = lmin < best_ref[...]
    besti_ref[...] = jnp.where(take, larg, besti_ref[...])
    best_ref[...] = jnp.where(take, lmin, best_ref[...])

    @pl.when(j == NJ - 1)
    def _():
        i = pl.program_id(0)
        idx_ref[0, :, :] = besti_ref[...]

        @pl.when(i == 0)
        def _():
            loss_ref[0, 0] = 0.0

        loss_ref[0, 0] += jnp.sum(best_ref[...])


def _vq_argmin(zf, cbt):
    return pl.pallas_call(
        _vq_body,
        grid=(NI, NJ),
        in_specs=[
            pl.BlockSpec((TN, D_EMB), lambda i, j: (i, 0)),
            pl.BlockSpec((D_EMB, TK), lambda i, j: (0, j)),
        ],
        out_specs=[
            pl.BlockSpec((1, TN, 1), lambda i, j: (i, 0, 0)),
            pl.BlockSpec(memory_space=pltpu.SMEM),
        ],
        out_shape=[
            jax.ShapeDtypeStruct((NI, TN, 1), jnp.int32),
            jax.ShapeDtypeStruct((1, 1), jnp.float32),
        ],
        scratch_shapes=[
            pltpu.VMEM((TN, 1), jnp.float32),
            pltpu.VMEM((TN, 1), jnp.int32),
        ],
    )(zf, cbt)


_NC, _NS = 2, 16
_NW = _NC * _NS
_BPW = N_TOK // _NW


@functools.cache
def _make_sc_gather():
    @functools.partial(
        pl.kernel,
        out_type=jax.ShapeDtypeStruct((N_TOK, D_EMB), jnp.float32),
        mesh=plsc.VectorSubcoreMesh(core_axis_name="c", subcore_axis_name="s"),
        scratch_types=[
            pltpu.VMEM((_BPW,), jnp.int32),
            pltpu.VMEM((_BPW, D_EMB), jnp.float32),
            pltpu.SemaphoreType.DMA,
        ],
        compiler_params=pltpu.CompilerParams(use_tc_tiling_on_sc=False),
    )
    def _sc_gather(table_hbm, idx_hbm, out_hbm, idx_v, rows_v, sem):
        wid = lax.axis_index("s") * _NC + lax.axis_index("c")
        base = wid * _BPW
        pltpu.sync_copy(idx_hbm.at[pl.ds(base, _BPW)], idx_v)
        pltpu.async_copy(table_hbm.at[idx_v], rows_v, sem).wait()
        pltpu.sync_copy(rows_v, out_hbm.at[pl.ds(base, _BPW)])

    return _sc_gather


def kernel(x, e1_w, e1_b, e2_w, e2_b, e3_w, e3_b, codebook,
           d1_w, d1_b, dt1_w, dt1_b, dt2_w, dt2_b,
           c1_w, c1_b, c2_w, c2_b, c3_w, c3_b, c4_w, c4_b, c5_w, c5_b):
    h = jax.nn.leaky_relu(_conv(x, e1_w, e1_b, 2, 1), 0.01)
    h = jax.nn.leaky_relu(_conv(h, e2_w, e2_b, 2, 1), 0.01)
    z = _conv(h, e3_w, e3_b, 1, 0)
    B, D, H, W = z.shape

    zf = jnp.transpose(z, (0, 2, 3, 1)).reshape(-1, D)
    idx3, loss_sum = _vq_argmin(zf, codebook.T)
    idx = idx3.reshape(-1)
    qf = _make_sc_gather()(codebook, idx)
    vq_loss = 1.25 * (loss_sum[0, 0] / (N_TOK * D_EMB))
    quantized = jnp.transpose(qf.reshape(B, H, W, D), (0, 3, 1, 2))

    d = jax.nn.leaky_relu(_conv(quantized, d1_w, d1_b, 1, 1), 0.01)
    d = jax.nn.leaky_relu(_convT(d, dt1_w, dt1_b), 0.01)
    x_recon = jnp.tanh(_convT(d, dt2_w, dt2_b))

    g = jax.nn.leaky_relu(_conv(x_recon, c1_w, c1_b, 2, 1), 0.2)
    g = jax.nn.leaky_relu(_conv(g, c2_w, c2_b, 2, 1), 0.2)
    g = jax.nn.leaky_relu(_conv(g, c3_w, c3_b, 2, 1), 0.2)
    g = jax.nn.leaky_relu(_conv(g, c4_w, c4_b, 1, 1), 0.2)
    disc = _conv(g, c5_w, c5_b, 1, 1)
    return (vq_loss, quantized, x_recon, disc)


# speedup vs baseline: 1.0088x; 1.0088x over previous
import functools

import jax
import jax.numpy as jnp
from jax import lax
from jax.experimental import pallas as pl
from jax.experimental.pallas import tpu as pltpu
from jax.experimental.pallas import tpu_sc as plsc

DN = ('NCHW', 'OIHW', 'NCHW')

N_TOK = 4096
D_EMB = 64
K_CB = 8192
TN = 256
TK = 8192
NI = N_TOK // TN
NJ = K_CB // TK


def _conv(x, w, b, stride, pad):
    y = lax.conv_general_dilated(x, w, (stride, stride),
                                 ((pad, pad), (pad, pad)),
                                 dimension_numbers=DN)
    return y + b[None, :, None, None]


def _convT(x, w, b):
    y = lax.conv_transpose(x, w, (2, 2), 'SAME', dimension_numbers=DN)
    return y + b[None, :, None, None]


def _vq_body(zf_ref, cbt_ref, idx_ref, loss_ref, best_ref, besti_ref):
    j = pl.program_id(1)

    @pl.when(j == 0)
    def _():
        best_ref[...] = jnp.full((TN, 1), jnp.inf, jnp.float32)
        besti_ref[...] = jnp.zeros((TN, 1), jnp.int32)

    zf = zf_ref[...]
    cbt = cbt_ref[...]
    zn = jnp.sum(zf * zf, axis=1, keepdims=True)
    cn = jnp.sum(cbt * cbt, axis=0, keepdims=True)
    mm2 = lax.dot_general(zf * (-2.0), cbt, (((1,), (0,)), ((), ())),
                          preferred_element_type=jnp.float32)
    d = (zn + cn) + mm2
    lmin = jnp.min(d, axis=1, keepdims=True)
    ids = lax.broadcasted_iota(jnp.int32, (TN, TK), 1).astype(jnp.float32)
    larg_f = jnp.min(jnp.where(d == lmin, ids, float(K_CB)), axis=1,
                     keepdims=True)
    larg = larg_f.astype(jnp.int32) + j * TK
    take = lmin < best_ref[...]
    besti_ref[...] = jnp.where(take, larg, besti_ref[...])
    best_ref[...] = jnp.where(take, lmin, best_ref[...])

    @pl.when(j == NJ - 1)
    def _():
        i = pl.program_id(0)
        idx_ref[0, :, :] = besti_ref[...]

        @pl.when(i == 0)
        def _():
            loss_ref[0, 0] = 0.0

        loss_ref[0, 0] += jnp.sum(best_ref[...])


def _vq_argmin(zf, cbt):
    return pl.pallas_call(
        _vq_body,
        grid=(NI, NJ),
        in_specs=[
            pl.BlockSpec((TN, D_EMB), lambda i, j: (i, 0)),
            pl.BlockSpec((D_EMB, TK), lambda i, j: (0, j)),
        ],
        out_specs=[
            pl.BlockSpec((1, TN, 1), lambda i, j: (i, 0, 0)),
            pl.BlockSpec(memory_space=pltpu.SMEM),
        ],
        out_shape=[
            jax.ShapeDtypeStruct((NI, TN, 1), jnp.int32),
            jax.ShapeDtypeStruct((1, 1), jnp.float32),
        ],
        scratch_shapes=[
            pltpu.VMEM((TN, 1), jnp.float32),
            pltpu.VMEM((TN, 1), jnp.int32),
        ],
    )(zf, cbt)


_NC, _NS = 2, 16
_NW = _NC * _NS
_BPW = N_TOK // _NW


@functools.cache
def _make_sc_gather():
    @functools.partial(
        pl.kernel,
        out_type=jax.ShapeDtypeStruct((N_TOK, D_EMB), jnp.float32),
        mesh=plsc.VectorSubcoreMesh(core_axis_name="c", subcore_axis_name="s"),
        scratch_types=[
            pltpu.VMEM((_BPW,), jnp.int32),
            pltpu.VMEM((_BPW, D_EMB), jnp.float32),
            pltpu.SemaphoreType.DMA,
        ],
        compiler_params=pltpu.CompilerParams(use_tc_tiling_on_sc=False),
    )
    def _sc_gather(table_hbm, idx_hbm, out_hbm, idx_v, rows_v, sem):
        wid = lax.axis_index("s") * _NC + lax.axis_index("c")
        base = wid * _BPW
        pltpu.sync_copy(idx_hbm.at[pl.ds(base, _BPW)], idx_v)
        pltpu.async_copy(table_hbm.at[idx_v], rows_v, sem).wait()
        pltpu.sync_copy(rows_v, out_hbm.at[pl.ds(base, _BPW)])

    return _sc_gather


def kernel(x, e1_w, e1_b, e2_w, e2_b, e3_w, e3_b, codebook,
           d1_w, d1_b, dt1_w, dt1_b, dt2_w, dt2_b,
           c1_w, c1_b, c2_w, c2_b, c3_w, c3_b, c4_w, c4_b, c5_w, c5_b):
    h = jax.nn.leaky_relu(_conv(x, e1_w, e1_b, 2, 1), 0.01)
    h = jax.nn.leaky_relu(_conv(h, e2_w, e2_b, 2, 1), 0.01)
    z = _conv(h, e3_w, e3_b, 1, 0)
    B, D, H, W = z.shape

    zf = jnp.transpose(z, (0, 2, 3, 1)).reshape(-1, D)
    idx3, loss_sum = _vq_argmin(zf, codebook.T)
    idx = idx3.reshape(-1)
    qf = _make_sc_gather()(codebook, idx)
    vq_loss = 1.25 * (loss_sum[0, 0] / (N_TOK * D_EMB))
    quantized = jnp.transpose(qf.reshape(B, H, W, D), (0, 3, 1, 2))

    d = jax.nn.leaky_relu(_conv(quantized, d1_w, d1_b, 1, 1), 0.01)
    d = jax.nn.leaky_relu(_convT(d, dt1_w, dt1_b), 0.01)
    x_recon = jnp.tanh(_convT(d, dt2_w, dt2_b))

    g = jax.nn.leaky_relu(_conv(x_recon, c1_w, c1_b, 2, 1), 0.2)
    g = jax.nn.leaky_relu(_conv(g, c2_w, c2_b, 2, 1), 0.2)
    g = jax.nn.leaky_relu(_conv(g, c3_w, c3_b, 2, 1), 0.2)
    g = jax.nn.leaky_relu(_conv(g, c4_w, c4_b, 1, 1), 0.2)
    disc = _conv(g, c5_w, c5_b, 1, 1)
    return (vq_loss, quantized, x_recon, disc)


# speedup vs baseline: 1.0214x; 1.0125x over previous
import functools

import jax
import jax.numpy as jnp
from jax import lax
from jax.experimental import pallas as pl
from jax.experimental.pallas import tpu as pltpu
from jax.experimental.pallas import tpu_sc as plsc

DN = ('NCHW', 'OIHW', 'NCHW')

N_TOK = 4096
D_EMB = 64
K_CB = 8192
TN = 512
TK = K_CB
NI = N_TOK // TN


def _conv(x, w, b, stride, pad):
    y = lax.conv_general_dilated(x, w, (stride, stride),
                                 ((pad, pad), (pad, pad)),
                                 dimension_numbers=DN)
    return y + b[None, :, None, None]


def _convT(x, w, b):
    y = lax.conv_transpose(x, w, (2, 2), 'SAME', dimension_numbers=DN)
    return y + b[None, :, None, None]


def _vq_body(zf_ref, cbt_ref, idx_ref, loss_ref):
    zf = zf_ref[...]
    cbt = cbt_ref[...]
    zn = jnp.sum(zf * zf, axis=1, keepdims=True)
    cn = jnp.sum(cbt * cbt, axis=0, keepdims=True)
    mm2 = lax.dot_general(zf * (-2.0), cbt, (((1,), (0,)), ((), ())),
                          preferred_element_type=jnp.float32)
    d = (zn + cn) + mm2
    lmin = jnp.min(d, axis=1, keepdims=True)
    ids = lax.broadcasted_iota(jnp.int32, (TN, TK), 1).astype(jnp.float32)
    larg_f = jnp.min(jnp.where(d == lmin, ids, float(K_CB)), axis=1,
                     keepdims=True)
    idx_ref[0, :, :] = larg_f.astype(jnp.int32)

    @pl.when(pl.program_id(0) == 0)
    def _():
        loss_ref[0, 0] = 0.0

    loss_ref[0, 0] += jnp.sum(lmin)


def _vq_argmin(zf, cbt):
    return pl.pallas_call(
        _vq_body,
        grid=(NI,),
        in_specs=[
            pl.BlockSpec((TN, D_EMB), lambda i: (i, 0)),
            pl.BlockSpec((D_EMB, TK), lambda i: (0, 0)),
        ],
        out_specs=[
            pl.BlockSpec((1, TN, 1), lambda i: (i, 0, 0)),
            pl.BlockSpec(memory_space=pltpu.SMEM),
        ],
        out_shape=[
            jax.ShapeDtypeStruct((NI, TN, 1), jnp.int32),
            jax.ShapeDtypeStruct((1, 1), jnp.float32),
        ],
    )(zf, cbt)


_NC, _NS = 2, 16
_NW = _NC * _NS
_BPW = N_TOK // _NW


@functools.cache
def _make_sc_gather():
    @functools.partial(
        pl.kernel,
        out_type=jax.ShapeDtypeStruct((N_TOK, D_EMB), jnp.float32),
        mesh=plsc.VectorSubcoreMesh(core_axis_name="c", subcore_axis_name="s"),
        scratch_types=[
            pltpu.VMEM((_BPW,), jnp.int32),
            pltpu.VMEM((_BPW, D_EMB), jnp.float32),
            pltpu.SemaphoreType.DMA,
        ],
        compiler_params=pltpu.CompilerParams(use_tc_tiling_on_sc=False),
    )
    def _sc_gather(table_hbm, idx_hbm, out_hbm, idx_v, rows_v, sem):
        wid = lax.axis_index("s") * _NC + lax.axis_index("c")
        base = wid * _BPW
        pltpu.sync_copy(idx_hbm.at[pl.ds(base, _BPW)], idx_v)
        pltpu.async_copy(table_hbm.at[idx_v], rows_v, sem).wait()
        pltpu.sync_copy(rows_v, out_hbm.at[pl.ds(base, _BPW)])

    return _sc_gather


def kernel(x, e1_w, e1_b, e2_w, e2_b, e3_w, e3_b, codebook,
           d1_w, d1_b, dt1_w, dt1_b, dt2_w, dt2_b,
           c1_w, c1_b, c2_w, c2_b, c3_w, c3_b, c4_w, c4_b, c5_w, c5_b):
    h = jax.nn.leaky_relu(_conv(x, e1_w, e1_b, 2, 1), 0.01)
    h = jax.nn.leaky_relu(_conv(h, e2_w, e2_b, 2, 1), 0.01)
    z = _conv(h, e3_w, e3_b, 1, 0)
    B, D, H, W = z.shape

    zf = jnp.transpose(z, (0, 2, 3, 1)).reshape(-1, D)
    idx3, loss_sum = _vq_argmin(zf, codebook.T)
    idx = idx3.reshape(-1)
    qf = _make_sc_gather()(codebook, idx)
    vq_loss = 1.25 * (loss_sum[0, 0] / (N_TOK * D_EMB))
    quantized = jnp.transpose(qf.reshape(B, H, W, D), (0, 3, 1, 2))

    d = jax.nn.leaky_relu(_conv(quantized, d1_w, d1_b, 1, 1), 0.01)
    d = jax.nn.leaky_relu(_convT(d, dt1_w, dt1_b), 0.01)
    x_recon = jnp.tanh(_convT(d, dt2_w, dt2_b))

    g = jax.nn.leaky_relu(_conv(x_recon, c1_w, c1_b, 2, 1), 0.2)
    g = jax.nn.leaky_relu(_conv(g, c2_w, c2_b, 2, 1), 0.2)
    g = jax.nn.leaky_relu(_conv(g, c3_w, c3_b, 2, 1), 0.2)
    g = jax.nn.leaky_relu(_conv(g, c4_w, c4_b, 1, 1), 0.2)
    disc = _conv(g, c5_w, c5_b, 1, 1)
    return (vq_loss, quantized, x_recon, disc)
